# baseline (speedup 1.0000x reference)
import functools

import jax
import jax.numpy as jnp
from jax import lax
from jax.experimental import pallas as pl
from jax.experimental.pallas import tpu as pltpu
from jax.experimental.pallas import tpu_sc as plsc

N = 100000
D = 128
NUM_SEGMENTS = 4096

NC = 2
NS = 16
NW = NC * NS
L = 16
SEG_PER_W = NUM_SEGMENTS // NW
CHUNK = 256
GPC = CHUNK // L
BLKS = 4
UNR = 8

_mesh = plsc.VectorSubcoreMesh(
    core_axis_name="c", subcore_axis_name="s", num_cores=NC, num_subcores=NS
)


@functools.partial(
    pl.kernel,
    out_type=jax.ShapeDtypeStruct((NUM_SEGMENTS, D), jnp.float32),
    mesh=_mesh,
    compiler_params=pltpu.CompilerParams(needs_layout_passes=False),
    scratch_types=[
        pltpu.VMEM((CHUNK * D,), jnp.float32),
        pltpu.VMEM((CHUNK * D,), jnp.float32),
        pltpu.VMEM((CHUNK + 16,), jnp.int32),
        pltpu.VMEM((CHUNK + 16,), jnp.int32),
        pltpu.VMEM((SEG_PER_W, D), jnp.float32),
        pltpu.VMEM((CHUNK,), jnp.float32),
        pltpu.VMEM((16,), jnp.float32),
        pltpu.VMEM((144,), jnp.float32),
        pltpu.VMEM((64,), jnp.int32),
        pltpu.SemaphoreType.DMA,
        pltpu.SemaphoreType.DMA,
        pltpu.SemaphoreType.DMA,
        pltpu.SemaphoreType.DMA,
    ],
)
def _wsum_sc(feats_hbm, ids_hbm, params_hbm, starts_hbm, out_hbm,
             feats_b0, feats_b1, ids_b0, ids_b1, acc, pbuf, sbuf, params_v,
             starts_v, semf0, semf1, semi0, semi1):
    cid = lax.axis_index("c")
    sid = lax.axis_index("s")
    wid = sid * NC + cid
    seg_base = wid * SEG_PER_W

    pltpu.sync_copy(params_hbm, params_v)
    pltpu.sync_copy(starts_hbm, starts_v)

    b_vec = params_v[pl.ds(D, 16)]
    w_vecs = [params_v[pl.ds(16 * j, 16)] for j in range(D // L)]

    r0 = starts_v[pl.ds(wid, 16)][0]
    r1 = starts_v[pl.ds(wid + 1, 16)][0]

    lane_iota = lax.iota(jnp.int32, L)
    segb_v = lax.broadcast(seg_base, (L,))
    rot_store = [16 * k + ((lane_iota + k) & 15) for k in range(L)]
    rot_load = [lane_iota * 16 + ((lane_iota + l) & 15) for l in range(L)]
    sig_bcast = [lax.broadcast(k, (L,)) for k in range(L)]

    zv = jnp.zeros((L,), jnp.float32)

    def zero_body(i, carry):
        for u in range(8):
            acc[i, pl.ds(u * L, 16)] = zv
        return carry

    lax.fori_loop(0, SEG_PER_W, zero_body, 0)

    a0 = (r0 // 8) * 8
    nchunks = (r1 - a0 + CHUNK - 1) // CHUNK

    slots = (
        (feats_b0, ids_b0, semf0, semi0),
        (feats_b1, ids_b1, semf1, semi1),
    )

    def chunk_start(k):
        return jnp.minimum(a0 + k * CHUNK, N - CHUNK)

    def copies(k, slot):
        cs = chunk_start(k)
        fbuf, ibuf, semf, semi = slots[slot]
        return (
            pltpu.make_async_copy(
                feats_hbm.at[pl.ds(cs * D, CHUNK * D)], fbuf, semf
            ),
            pltpu.make_async_copy(
                ids_hbm.at[pl.ds(cs, CHUNK)], ibuf.at[pl.ds(0, CHUNK)], semi
            ),
        )

    def issue(k, slot):
        for c in copies(k, slot):
            c.start()

    def wait(k, slot):
        for c in copies(k, slot):
            c.wait()

    def process(k, slot):
        fbuf, ibuf, _, _ = slots[slot]
        cs_u = a0 + k * CHUNK
        cs = chunk_start(k)
        p_lo = jnp.maximum(r0, cs_u)
        p_hi = jnp.minimum(r1, cs_u + CHUNK)
        bl_lo = p_lo - cs
        bl_hi = p_hi - cs
        g_lo = bl_lo // L
        g_hi = (bl_hi + L - 1) // L
        lo_v = lax.broadcast(bl_lo, (L,))
        hi_v = lax.broadcast(bl_hi, (L,))

        def group_body(g, c2):
            gb = g * L
            idv = ibuf[pl.ds(gb, 16)]
            lsegv = jnp.minimum(jnp.maximum(idv - segb_v, 0), SEG_PER_W - 1)
            for kk in range(L):
                base = (gb + kk) * D
                part = fbuf[pl.ds(base, 16)] * w_vecs[0]
                for j in range(1, D // L):
                    part = part + fbuf[pl.ds(base + 16 * j, 16)] * w_vecs[j]
                plsc.store_scatter(pbuf, [rot_store[kk]], part)
            tot = plsc.load_gather(pbuf, [rot_load[0]])
            for l in range(1, L):
                tot = tot + plsc.load_gather(pbuf, [rot_load[l]])
            rv = lax.broadcast(gb, (L,)) + lane_iota
            maskv = (rv >= lo_v) & (rv < hi_v)
            sigv = 1.0 / (1.0 + jnp.exp(-(tot + b_vec)))
            sigv = jnp.where(maskv, sigv, 0.0)
            for c in range(L):
                plsc.store_scatter(pbuf, [rot_store[c]], sigv)
            for kk in range(L):
                base = (gb + kk) * D
                sgk = plsc.load_gather(pbuf, [rot_load[kk]])
                lseg = lsegv[kk]
                for j in range(D // L):
                    plsc.addupdate(
                        acc.at[lseg, pl.ds(16 * j, 16)],
                        fbuf[pl.ds(base + 16 * j, 16)] * sgk,
                    )
            return c2

        lax.fori_loop(g_lo, g_hi, group_body, 0)

    @pl.when(nchunks > 0)
    def _():
        issue(0, 0)

    def pair_body(kk, carry):
        k0 = 2 * kk
        k1 = k0 + 1

        @pl.when(k0 < nchunks)
        def _():
            wait(k0, 0)

            @pl.when(k1 < nchunks)
            def _():
                issue(k1, 1)

            process(k0, 0)

        @pl.when(k1 < nchunks)
        def _():
            wait(k1, 1)

            @pl.when(k1 + 1 < nchunks)
            def _():
                issue(k1 + 1, 0)

            process(k1, 1)

        return carry

    lax.fori_loop(0, (nchunks + 1) // 2, pair_body, 0)

    pltpu.sync_copy(acc, out_hbm.at[pl.ds(seg_base, SEG_PER_W)])


def kernel(feats, segment_ids, W, b):
    ids32 = segment_ids.astype(jnp.int32)
    bounds = jnp.arange(0, NUM_SEGMENTS + 1, SEG_PER_W, dtype=jnp.int32)
    starts = jnp.searchsorted(ids32, bounds, side="left").astype(jnp.int32)
    starts_p = jnp.zeros((64,), jnp.int32).at[: NW + 1].set(starts)
    params = jnp.concatenate(
        [W.reshape(D).astype(jnp.float32), jnp.full((16,), b[0], jnp.float32)]
    )
    return _wsum_sc(feats.reshape(-1), ids32, params, starts_p)


# speedup vs baseline: 1.4722x; 1.4722x over previous
import functools

import jax
import jax.numpy as jnp
from jax import lax
from jax.experimental import pallas as pl
from jax.experimental.pallas import tpu as pltpu
from jax.experimental.pallas import tpu_sc as plsc

N = 100000
D = 128
NUM_SEGMENTS = 4096

NC = 2
NS = 16
NW = NC * NS
L = 16
VPR = D // L
SEG_PER_W = NUM_SEGMENTS // NW
CHUNK = 256

_mesh = plsc.VectorSubcoreMesh(
    core_axis_name="c", subcore_axis_name="s", num_cores=NC, num_subcores=NS
)


@functools.partial(
    pl.kernel,
    out_type=jax.ShapeDtypeStruct((NUM_SEGMENTS, D), jnp.float32),
    mesh=_mesh,
    compiler_params=pltpu.CompilerParams(needs_layout_passes=False),
    scratch_types=[
        pltpu.VMEM((CHUNK, D), jnp.float32),
        pltpu.VMEM((CHUNK, D), jnp.float32),
        pltpu.VMEM((CHUNK + 16,), jnp.int32),
        pltpu.VMEM((CHUNK + 16,), jnp.int32),
        pltpu.VMEM((SEG_PER_W, D), jnp.float32),
        pltpu.VMEM((144,), jnp.float32),
        pltpu.VMEM((64,), jnp.int32),
        pltpu.SemaphoreType.DMA,
        pltpu.SemaphoreType.DMA,
        pltpu.SemaphoreType.DMA,
        pltpu.SemaphoreType.DMA,
    ],
)
def _wsum_sc(feats_hbm, ids_hbm, params_hbm, starts_hbm, out_hbm,
             feats_b0, feats_b1, ids_b0, ids_b1, acc, params_v, starts_v,
             semf0, semf1, semi0, semi1):
    cid = lax.axis_index("c")
    sid = lax.axis_index("s")
    wid = sid * NC + cid
    seg_base = wid * SEG_PER_W

    pltpu.sync_copy(params_hbm, params_v)
    pltpu.sync_copy(starts_hbm, starts_v)

    w_vecs = [params_v[pl.ds(16 * j, 16)] for j in range(VPR)]
    b_vec = params_v[pl.ds(D, 16)]

    r0 = starts_v[pl.ds(wid, 16)][0]
    r1 = starts_v[pl.ds(wid + 1, 16)][0]

    lane_iota = lax.iota(jnp.int32, L)
    low_half = lane_iota < 8

    zv = jnp.zeros((L,), jnp.float32)

    def zero_body(i, carry):
        for j in range(VPR):
            acc[i, pl.ds(16 * j, 16)] = zv
        return carry

    lax.fori_loop(0, SEG_PER_W, zero_body, 0)

    a0 = (r0 // 8) * 8
    nchunks = (r1 - a0 + CHUNK - 1) // CHUNK

    slots = (
        (feats_b0, ids_b0, semf0, semi0),
        (feats_b1, ids_b1, semf1, semi1),
    )

    def chunk_start(k):
        return jnp.minimum(a0 + k * CHUNK, N - CHUNK)

    def copies(k, slot):
        cs = chunk_start(k)
        fbuf, ibuf, semf, semi = slots[slot]
        return (
            pltpu.make_async_copy(feats_hbm.at[pl.ds(cs, CHUNK)], fbuf, semf),
            pltpu.make_async_copy(
                ids_hbm.at[pl.ds(cs, CHUNK)], ibuf.at[pl.ds(0, CHUNK)], semi
            ),
        )

    def issue(k, slot):
        for c in copies(k, slot):
            c.start()

    def wait(k, slot):
        for c in copies(k, slot):
            c.wait()

    def process(k, slot):
        fbuf, ibuf, _, _ = slots[slot]
        cs_u = a0 + k * CHUNK
        cs = chunk_start(k)
        p_lo = jnp.maximum(r0, cs_u)
        p_hi = jnp.minimum(r1, cs_u + CHUNK)

        def load_row(li):
            return [fbuf[li, pl.ds(16 * j, 16)] for j in range(VPR)]

        def fold(row):
            part = row[0] * w_vecs[0]
            for j in range(1, VPR):
                part = part + row[j] * w_vecs[j]
            return part + lax.rev(part, (0,))

        def accumulate(li, row, sig):
            seg = ibuf[pl.ds(li, 16)][0]
            lseg = seg - seg_base
            for j in range(VPR):
                plsc.addupdate(acc.at[lseg, pl.ds(16 * j, 16)], row[j] * sig)

        npairs = (p_hi - p_lo) // 2

        def pair_body(t, c2):
            ra = p_lo + 2 * t - cs
            rb = ra + 1
            rowa = load_row(ra)
            rowb = load_row(rb)
            packed = jnp.where(low_half, fold(rowa), fold(rowb))
            cum = plsc.cumsum(packed)
            sa = cum[7]
            sab = cum[15]
            siga = 1.0 / (1.0 + jnp.exp(-(lax.broadcast(sa, (L,)) + b_vec)))
            sigb = 1.0 / (
                1.0 + jnp.exp(-(lax.broadcast(sab - sa, (L,)) + b_vec))
            )
            accumulate(ra, rowa, siga)
            accumulate(rb, rowb, sigb)
            return c2

        lax.fori_loop(0, npairs, pair_body, 0)

        @pl.when(p_lo + 2 * npairs < p_hi)
        def _():
            li = p_lo + 2 * npairs - cs
            row = load_row(li)
            cum = plsc.cumsum(fold(row))
            sig = 1.0 / (
                1.0 + jnp.exp(-(lax.broadcast(cum[7], (L,)) + b_vec))
            )
            accumulate(li, row, sig)

    @pl.when(nchunks > 0)
    def _():
        issue(0, 0)

    def pair_body(kk, carry):
        k0 = 2 * kk
        k1 = k0 + 1

        @pl.when(k0 < nchunks)
        def _():
            wait(k0, 0)

            @pl.when(k1 < nchunks)
            def _():
                issue(k1, 1)

            process(k0, 0)

        @pl.when(k1 < nchunks)
        def _():
            wait(k1, 1)

            @pl.when(k1 + 1 < nchunks)
            def _():
                issue(k1 + 1, 0)

            process(k1, 1)

        return carry

    lax.fori_loop(0, (nchunks + 1) // 2, pair_body, 0)

    pltpu.sync_copy(acc, out_hbm.at[pl.ds(seg_base, SEG_PER_W)])


def kernel(feats, segment_ids, W, b):
    ids32 = segment_ids.astype(jnp.int32)
    bounds = jnp.arange(0, NUM_SEGMENTS + 1, SEG_PER_W, dtype=jnp.int32)
    starts = jnp.searchsorted(ids32, bounds, side="left").astype(jnp.int32)
    starts_p = jnp.zeros((64,), jnp.int32).at[: NW + 1].set(starts)
    params = jnp.concatenate(
        [W.reshape(D).astype(jnp.float32), jnp.full((16,), b[0], jnp.float32)]
    )
    return _wsum_sc(feats, ids32, params, starts_p)


# speedup vs baseline: 1.5880x; 1.0787x over previous
import functools

import jax
import jax.numpy as jnp
from jax import lax
from jax.experimental import pallas as pl
from jax.experimental.pallas import tpu as pltpu
from jax.experimental.pallas import tpu_sc as plsc

N = 100000
D = 128
NUM_SEGMENTS = 4096

NC = 2
NS = 16
NW = NC * NS
L = 16
VPR = D // L
SEG_PER_W = NUM_SEGMENTS // NW
CHUNK = 384

_mesh = plsc.VectorSubcoreMesh(
    core_axis_name="c", subcore_axis_name="s", num_cores=NC, num_subcores=NS
)


@functools.partial(
    pl.kernel,
    out_type=jax.ShapeDtypeStruct((NUM_SEGMENTS, D), jnp.float32),
    mesh=_mesh,
    compiler_params=pltpu.CompilerParams(needs_layout_passes=False),
    scratch_types=[
        pltpu.VMEM((CHUNK, D), jnp.float32),
        pltpu.VMEM((CHUNK, D), jnp.float32),
        pltpu.VMEM((CHUNK + 16,), jnp.int32),
        pltpu.VMEM((CHUNK + 16,), jnp.int32),
        pltpu.VMEM((SEG_PER_W, D), jnp.float32),
        pltpu.VMEM((144,), jnp.float32),
        pltpu.VMEM((64,), jnp.int32),
        pltpu.SemaphoreType.DMA,
        pltpu.SemaphoreType.DMA,
        pltpu.SemaphoreType.DMA,
        pltpu.SemaphoreType.DMA,
    ],
)
def _wsum_sc(feats_hbm, ids_hbm, params_hbm, starts_hbm, out_hbm,
             feats_b0, feats_b1, ids_b0, ids_b1, acc, params_v, starts_v,
             semf0, semf1, semi0, semi1):
    cid = lax.axis_index("c")
    sid = lax.axis_index("s")
    wid = sid * NC + cid
    seg_base = wid * SEG_PER_W

    pltpu.sync_copy(params_hbm, params_v)
    pltpu.sync_copy(starts_hbm, starts_v)

    w_vecs = [params_v[pl.ds(16 * j, 16)] for j in range(VPR)]
    b_vec = params_v[pl.ds(D, 16)]

    r0 = starts_v[pl.ds(wid, 16)][0]
    r1 = starts_v[pl.ds(wid + 1, 16)][0]

    lane_iota = lax.iota(jnp.int32, L)
    low_half = lane_iota < 8

    zv = jnp.zeros((L,), jnp.float32)

    def zero_body(i, carry):
        for j in range(VPR):
            acc[i, pl.ds(16 * j, 16)] = zv
        return carry

    lax.fori_loop(0, SEG_PER_W, zero_body, 0)

    a0 = (r0 // 8) * 8
    nchunks = (r1 - a0 + CHUNK - 1) // CHUNK

    slots = (
        (feats_b0, ids_b0, semf0, semi0),
        (feats_b1, ids_b1, semf1, semi1),
    )

    def chunk_start(k):
        return jnp.minimum(a0 + k * CHUNK, N - CHUNK)

    def copies(k, slot):
        cs = chunk_start(k)
        fbuf, ibuf, semf, semi = slots[slot]
        return (
            pltpu.make_async_copy(feats_hbm.at[pl.ds(cs, CHUNK)], fbuf, semf),
            pltpu.make_async_copy(
                ids_hbm.at[pl.ds(cs, CHUNK)], ibuf.at[pl.ds(0, CHUNK)], semi
            ),
        )

    def issue(k, slot):
        for c in copies(k, slot):
            c.start()

    def wait(k, slot):
        for c in copies(k, slot):
            c.wait()

    def process(k, slot):
        fbuf, ibuf, _, _ = slots[slot]
        cs_u = a0 + k * CHUNK
        cs = chunk_start(k)
        p_lo = jnp.maximum(r0, cs_u)
        p_hi = jnp.minimum(r1, cs_u + CHUNK)

        def load_row(li):
            return [fbuf[li, pl.ds(16 * j, 16)] for j in range(VPR)]

        def fold(row):
            part = row[0] * w_vecs[0]
            for j in range(1, VPR):
                part = part + row[j] * w_vecs[j]
            return part + lax.rev(part, (0,))

        def accumulate(seg, row, sig):
            lseg = seg - seg_base
            for j in range(VPR):
                plsc.addupdate(acc.at[lseg, pl.ds(16 * j, 16)], row[j] * sig)

        npairs = (p_hi - p_lo) // 2

        def pair_body(t, c2):
            ra = p_lo + 2 * t - cs
            rb = ra + 1
            iv = ibuf[pl.ds(ra, 16)]
            rowa = load_row(ra)
            rowb = load_row(rb)
            packed = jnp.where(low_half, fold(rowa), fold(rowb))
            cum = plsc.cumsum(packed)
            sa = cum[7]
            sab = cum[15]
            siga = 1.0 / (1.0 + jnp.exp(-(lax.broadcast(sa, (L,)) + b_vec)))
            sigb = 1.0 / (
                1.0 + jnp.exp(-(lax.broadcast(sab - sa, (L,)) + b_vec))
            )
            accumulate(iv[0], rowa, siga)
            accumulate(iv[1], rowb, sigb)
            return c2

        lax.fori_loop(0, npairs, pair_body, 0)

        @pl.when(p_lo + 2 * npairs < p_hi)
        def _():
            li = p_lo + 2 * npairs - cs
            row = load_row(li)
            cum = plsc.cumsum(fold(row))
            sig = 1.0 / (
                1.0 + jnp.exp(-(lax.broadcast(cum[7], (L,)) + b_vec))
            )
            accumulate(ibuf[pl.ds(li, 16)][0], row, sig)

    @pl.when(nchunks > 0)
    def _():
        issue(0, 0)

    def pair_body(kk, carry):
        k0 = 2 * kk
        k1 = k0 + 1

        @pl.when(k0 < nchunks)
        def _():
            wait(k0, 0)

            @pl.when(k1 < nchunks)
            def _():
                issue(k1, 1)

            process(k0, 0)

        @pl.when(k1 < nchunks)
        def _():
            wait(k1, 1)

            @pl.when(k1 + 1 < nchunks)
            def _():
                issue(k1 + 1, 0)

            process(k1, 1)

        return carry

    lax.fori_loop(0, (nchunks + 1) // 2, pair_body, 0)

    pltpu.sync_copy(acc, out_hbm.at[pl.ds(seg_base, SEG_PER_W)])


def kernel(feats, segment_ids, W, b):
    ids32 = segment_ids.astype(jnp.int32)
    bounds = jnp.arange(0, NUM_SEGMENTS + 1, SEG_PER_W, dtype=jnp.int32)
    starts = jnp.searchsorted(ids32, bounds, side="left").astype(jnp.int32)
    starts_p = jnp.zeros((64,), jnp.int32).at[: NW + 1].set(starts)
    params = jnp.concatenate(
        [W.reshape(D).astype(jnp.float32), jnp.full((16,), b[0], jnp.float32)]
    )
    return _wsum_sc(feats, ids32, params, starts_p)
